# baseline (speedup 1.0000x reference)
import functools

import jax
import jax.numpy as jnp
from jax import lax
from jax.experimental import pallas as pl
from jax.experimental.pallas import tpu as pltpu
from jax.experimental.pallas import tpu_sc as plsc

NC, NS = 2, 16
NW = NC * NS
CH = 128


def _xw_body(wc_ref, w_ref, x_ref, o_ref):
    r = pl.program_id(1)
    w = w_ref[...]
    wrel = wc_ref[r, 0] * w[0]
    for b in range(1, w.shape[0]):
        wrel = wrel + wc_ref[r, b] * w[b]
    o_ref[...] = jnp.dot(x_ref[...], wrel, preferred_element_type=jnp.float32)


def _add_body(p_ref, o_ref):
    o_ref[...] = p_ref[0] + p_ref[1]


def kernel(x, edge_index, edge_type, weight, w_comp):
    N, F = x.shape
    B, _, O = weight.shape
    R = w_comp.shape[0]
    E = edge_index.shape[1]

    BN = 2000
    NB = N // BN
    xw = pl.pallas_call(
        _xw_body,
        grid=(NB, R),
        in_specs=[
            pl.BlockSpec(memory_space=pltpu.SMEM),
            pl.BlockSpec((B, F, O), lambda n, r: (0, 0, 0)),
            pl.BlockSpec((BN, F), lambda n, r: (n, 0)),
        ],
        out_specs=pl.BlockSpec((BN, O), lambda n, r: (r * NB + n, 0)),
        out_shape=jax.ShapeDtypeStruct((R * N, O), jnp.float32),
    )(w_comp, weight, x)

    src = edge_index[0]
    dst = edge_index[1]
    gidx = edge_type * N + src
    NBUF = 2
    FAST_CORE = 0
    FAST_FRAC = 0.84
    EPP = -(-E // NS)
    N0 = NBUF * (-(-int(EPP * FAST_FRAC) // (CH * NBUF)))
    N1 = NBUF * (-(-(EPP - N0 * CH) // (CH * NBUF)))
    STAGE = 64
    LROWS = -(-max(N0, N1) // STAGE) * STAGE
    GB = (R * N - 1).bit_length()
    assert GB + N.bit_length() <= 31
    pad_val = jnp.int32(N << GB)
    packed = (dst << GB) | gidx
    e_pad = NS * (N0 + N1) * CH
    packed = jnp.concatenate([packed, jnp.full((e_pad - E,), pad_val)])
    blocks = []
    off = 0
    for n_c in ([N0, N1] if FAST_CORE == 0 else [N1, N0]):
        blk = packed[off:off + NS * n_c * CH].reshape(NS, n_c * CH)
        off += NS * n_c * CH
        fill = jnp.full((NS, (LROWS - n_c) * CH), pad_val)
        blocks.append(jnp.concatenate([blk, fill], axis=1))
    packed = jnp.stack(blocks).reshape(NW, LROWS, CH)

    ZR = (-(-(N + 1) // NS) + 7) // 8 * 8
    N_pad = ZR * NS

    mesh = plsc.VectorSubcoreMesh(
        core_axis_name="c", subcore_axis_name="s", num_cores=NC, num_subcores=NS
    )

    R0, R1 = N0 // NBUF, N1 // NBUF
    REFILL_K = (STAGE - NBUF) // NBUF

    @functools.partial(
        pl.kernel,
        out_type=jax.ShapeDtypeStruct((NC, N_pad, O), jnp.float32),
        mesh=mesh,
        scratch_types=[
            pltpu.VMEM((STAGE, CH), jnp.int32),
            pltpu.VMEM_SHARED((N_pad, O), jnp.float32),
        ]
        + [pltpu.VMEM((CH, O), jnp.float32)] * NBUF
        + [pltpu.VMEM((2, CH), jnp.int32)] * NBUF
        + [pltpu.SemaphoreType.DMA] * (2 * NBUF),
    )
    def sc_scatter(xw_hbm, idx_hbm, out_hbm, *s):
        slab, acc = s[0], s[1]
        rows = s[2:2 + NBUF]
        gd = s[2 + NBUF:2 + 2 * NBUF]
        gsem = s[2 + 2 * NBUF:2 + 3 * NBUF]
        ssem = s[2 + 3 * NBUF:2 + 4 * NBUF]
        cid = lax.axis_index("c")
        sid = lax.axis_index("s")
        wid = cid * NS + sid
        mask = jnp.int32((1 << GB) - 1)

        def unpack(ci, b):
            sr = lax.rem(ci, STAGE)
            for l in range(CH // 16):
                v = slab[sr, pl.ds(16 * l, 16)]
                gd[b][0, pl.ds(16 * l, 16)] = v & mask
                gd[b][1, pl.ds(16 * l, 16)] = lax.shift_right_logical(v, GB)

        pltpu.sync_copy(idx_hbm.at[wid, pl.ds(0, STAGE)], slab)
        zv = jnp.zeros((16,), jnp.float32)

        def zrow(i, carry):
            for l in range(O // 16):
                rows[0][i, pl.ds(16 * l, 16)] = zv
            return carry

        lax.fori_loop(0, CH, zrow, 0)
        nfull, rem_rows = ZR // CH, ZR % CH
        for z in range(nfull):
            pltpu.sync_copy(rows[0], acc.at[pl.ds(sid * ZR + z * CH, CH)])
        if rem_rows:
            pltpu.sync_copy(rows[0].at[pl.ds(0, rem_rows)],
                            acc.at[pl.ds(sid * ZR + nfull * CH, rem_rows)])
        kper = STAGE // NBUF

        def run(table, rounds):
            for b in range(NBUF):
                unpack(b, b)
                pltpu.async_copy(table.at[gd[b].at[0]], rows[b], gsem[b])
            plsc.subcore_barrier()

            def body(k, carry):
                @pl.when(jnp.logical_and(lax.rem(k, kper) == REFILL_K,
                                         k < rounds - 1))
                def _():
                    stg = k // kper + 1
                    pltpu.sync_copy(idx_hbm.at[wid, pl.ds(stg * STAGE, STAGE)],
                                    slab)

                for b in range(NBUF):
                    ci = k * NBUF + b
                    pltpu.make_async_copy(table.at[gd[b].at[0]],
                                          rows[b], gsem[b]).wait()
                    pltpu.sync_copy(rows[b], acc.at[gd[b].at[1]], add=True)

                    @pl.when(k < rounds - 1)
                    def _():
                        unpack(ci + NBUF, b)
                        pltpu.async_copy(table.at[gd[b].at[0]], rows[b],
                                         gsem[b])
                return carry

            lax.fori_loop(0, rounds, body, 0)

        @pl.when(cid == FAST_CORE)
        def _():
            run(xw_hbm, R0)

        @pl.when(cid != FAST_CORE)
        def _():
            run(xw_hbm, R1)

        plsc.subcore_barrier()
        pltpu.sync_copy(acc.at[pl.ds(sid * ZR, ZR)],
                        out_hbm.at[cid, pl.ds(sid * ZR, ZR)])

    partial = sc_scatter(xw, packed)

    BN2 = 1000
    out = pl.pallas_call(
        _add_body,
        grid=(N // BN2,),
        in_specs=[pl.BlockSpec((NC, BN2, O), lambda n: (0, n, 0))],
        out_specs=pl.BlockSpec((BN2, O), lambda n: (n, 0)),
        out_shape=jax.ShapeDtypeStruct((N, O), jnp.float32),
    )(partial)
    return out


# speedup vs baseline: 1.1429x; 1.1429x over previous
import functools

import jax
import jax.numpy as jnp
from jax import lax
from jax.experimental import pallas as pl
from jax.experimental.pallas import tpu as pltpu
from jax.experimental.pallas import tpu_sc as plsc

NC, NS = 2, 16
NW = NC * NS
CH = 128


def _xw_body(wc_ref, w_ref, x_ref, o_ref):
    r = pl.program_id(1)
    w = w_ref[...]
    wrel = wc_ref[r, 0] * w[0]
    for b in range(1, w.shape[0]):
        wrel = wrel + wc_ref[r, b] * w[b]
    o_ref[...] = jnp.dot(x_ref[...], wrel, preferred_element_type=jnp.float32)


def _add_body(p_ref, o_ref):
    o_ref[...] = p_ref[0] + p_ref[1]


def kernel(x, edge_index, edge_type, weight, w_comp):
    N, F = x.shape
    B, _, O = weight.shape
    R = w_comp.shape[0]
    E = edge_index.shape[1]

    BN = 2000
    NB = N // BN
    xw = pl.pallas_call(
        _xw_body,
        grid=(NB, R),
        in_specs=[
            pl.BlockSpec(memory_space=pltpu.SMEM),
            pl.BlockSpec((B, F, O), lambda n, r: (0, 0, 0)),
            pl.BlockSpec((BN, F), lambda n, r: (n, 0)),
        ],
        out_specs=pl.BlockSpec((BN, O), lambda n, r: (r * NB + n, 0)),
        out_shape=jax.ShapeDtypeStruct((R * N, O), jnp.float32),
    )(w_comp, weight, x)

    src = edge_index[0]
    dst = edge_index[1]
    gidx = edge_type * N + src
    NBUF = 2
    FAST_CORE = 0
    FAST_FRAC = 0.81
    EPP = -(-E // NS)
    N0 = NBUF * (-(-int(EPP * FAST_FRAC) // (CH * NBUF)))
    N1 = NBUF * (-(-(EPP - N0 * CH) // (CH * NBUF)))
    STAGE = 64
    GB = (R * N - 1).bit_length()
    assert GB + N.bit_length() <= 31
    pad_val = jnp.int32(N << GB)
    slack = (-(-N1 // STAGE) * STAGE - N1) * CH
    e_pad = NS * (N0 + N1) * CH + slack
    packed = (dst << GB) | gidx
    packed = jnp.concatenate([packed, jnp.full((e_pad - E,), pad_val)])

    ZR = (-(-(N + 1) // NS) + 7) // 8 * 8
    N_pad = ZR * NS

    mesh = plsc.VectorSubcoreMesh(
        core_axis_name="c", subcore_axis_name="s", num_cores=NC, num_subcores=NS
    )

    R0, R1 = N0 // NBUF, N1 // NBUF
    REFILL_K = (STAGE - NBUF) // NBUF

    @functools.partial(
        pl.kernel,
        out_type=jax.ShapeDtypeStruct((NC, N_pad, O), jnp.float32),
        mesh=mesh,
        scratch_types=[
            pltpu.VMEM((STAGE * CH,), jnp.int32),
            pltpu.VMEM_SHARED((N_pad, O), jnp.float32),
        ]
        + [pltpu.VMEM((CH, O), jnp.float32)] * NBUF
        + [pltpu.VMEM((2, CH), jnp.int32)] * NBUF
        + [pltpu.SemaphoreType.DMA] * (2 * NBUF),
    )
    def sc_scatter(xw_hbm, idx_hbm, out_hbm, *s):
        slab, acc = s[0], s[1]
        rows = s[2:2 + NBUF]
        gd = s[2 + NBUF:2 + 2 * NBUF]
        gsem = s[2 + 2 * NBUF:2 + 3 * NBUF]
        ssem = s[2 + 3 * NBUF:2 + 4 * NBUF]
        cid = lax.axis_index("c")
        sid = lax.axis_index("s")
        mask = jnp.int32((1 << GB) - 1)
        base_e = jnp.where(cid == FAST_CORE, sid * (N0 * CH),
                           NS * (N0 * CH) + sid * (N1 * CH))

        def unpack(ci, b):
            so = lax.rem(ci, STAGE) * CH
            for l in range(CH // 16):
                v = slab[pl.ds(so + 16 * l, 16)]
                gd[b][0, pl.ds(16 * l, 16)] = v & mask
                gd[b][1, pl.ds(16 * l, 16)] = lax.shift_right_logical(v, GB)

        pltpu.sync_copy(idx_hbm.at[pl.ds(base_e, STAGE * CH)], slab)
        zv = jnp.zeros((16,), jnp.float32)

        def zrow(i, carry):
            for l in range(O // 16):
                rows[0][i, pl.ds(16 * l, 16)] = zv
            return carry

        lax.fori_loop(0, CH, zrow, 0)
        nfull, rem_rows = ZR // CH, ZR % CH
        for z in range(nfull):
            pltpu.sync_copy(rows[0], acc.at[pl.ds(sid * ZR + z * CH, CH)])
        if rem_rows:
            pltpu.sync_copy(rows[0].at[pl.ds(0, rem_rows)],
                            acc.at[pl.ds(sid * ZR + nfull * CH, rem_rows)])
        kper = STAGE // NBUF

        def run(table, rounds):
            for b in range(NBUF):
                unpack(b, b)
                pltpu.async_copy(table.at[gd[b].at[0]], rows[b], gsem[b])
            plsc.subcore_barrier()

            def body(k, carry):
                @pl.when(jnp.logical_and(lax.rem(k, kper) == REFILL_K,
                                         k < rounds - 1))
                def _():
                    stg = k // kper + 1
                    pltpu.sync_copy(
                        idx_hbm.at[pl.ds(base_e + stg * (STAGE * CH),
                                         STAGE * CH)], slab)

                for b in range(NBUF):
                    ci = k * NBUF + b
                    pltpu.make_async_copy(table.at[gd[b].at[0]],
                                          rows[b], gsem[b]).wait()
                    pltpu.sync_copy(rows[b], acc.at[gd[b].at[1]], add=True)

                    @pl.when(k < rounds - 1)
                    def _():
                        unpack(ci + NBUF, b)
                        pltpu.async_copy(table.at[gd[b].at[0]], rows[b],
                                         gsem[b])
                return carry

            lax.fori_loop(0, rounds, body, 0)

        @pl.when(cid == FAST_CORE)
        def _():
            run(xw_hbm, R0)

        @pl.when(cid != FAST_CORE)
        def _():
            run(xw_hbm, R1)

        plsc.subcore_barrier()
        pltpu.sync_copy(acc.at[pl.ds(sid * ZR, ZR)],
                        out_hbm.at[cid, pl.ds(sid * ZR, ZR)])

    partial = sc_scatter(xw, packed)

    BN2 = 1000
    out = pl.pallas_call(
        _add_body,
        grid=(N // BN2,),
        in_specs=[pl.BlockSpec((NC, BN2, O), lambda n: (0, n, 0))],
        out_specs=pl.BlockSpec((BN2, O), lambda n: (n, 0)),
        out_shape=jax.ShapeDtypeStruct((N, O), jnp.float32),
    )(partial)
    return out


# speedup vs baseline: 1.1663x; 1.0205x over previous
import functools

import jax
import jax.numpy as jnp
from jax import lax
from jax.experimental import pallas as pl
from jax.experimental.pallas import tpu as pltpu
from jax.experimental.pallas import tpu_sc as plsc

NC, NS = 2, 16
NW = NC * NS
CH = 128


def _xw_body(wc_ref, w_ref, x_ref, o_ref):
    r = pl.program_id(1)
    w = w_ref[...]
    wrel = wc_ref[r, 0] * w[0]
    for b in range(1, w.shape[0]):
        wrel = wrel + wc_ref[r, b] * w[b]
    o_ref[...] = jnp.dot(x_ref[...], wrel, preferred_element_type=jnp.float32)


def _add_body(p_ref, o_ref):
    o_ref[...] = p_ref[0] + p_ref[1]


def _pack_body(n, gb, ei_ref, et_ref, o_ref):
    src = ei_ref[0]
    dst = ei_ref[1]
    o_ref[...] = (dst << gb) | (et_ref[...] * n + src)


def kernel(x, edge_index, edge_type, weight, w_comp):
    N, F = x.shape
    B, _, O = weight.shape
    R = w_comp.shape[0]
    E = edge_index.shape[1]

    BN = 2000
    NB = N // BN
    xw = pl.pallas_call(
        _xw_body,
        grid=(NB, R),
        in_specs=[
            pl.BlockSpec(memory_space=pltpu.SMEM),
            pl.BlockSpec((B, F, O), lambda n, r: (0, 0, 0)),
            pl.BlockSpec((BN, F), lambda n, r: (n, 0)),
        ],
        out_specs=pl.BlockSpec((BN, O), lambda n, r: (r * NB + n, 0)),
        out_shape=jax.ShapeDtypeStruct((R * N, O), jnp.float32),
    )(w_comp, weight, x)

    NBUF = 2
    FAST_CORE = 0
    FAST_FRAC = 0.81
    EPP = -(-E // NS)
    N0 = NBUF * (-(-int(EPP * FAST_FRAC) // (CH * NBUF)))
    N1 = NBUF * (-(-(EPP - N0 * CH) // (CH * NBUF)))
    STAGE = 64
    GB = (R * N - 1).bit_length()
    assert GB + N.bit_length() <= 31
    pad_val = jnp.int32(N << GB)
    slack = (-(-N1 // STAGE) * STAGE - N1) * CH
    e_pad = NS * (N0 + N1) * CH + slack
    EB = E // 128
    packed = pl.pallas_call(
        functools.partial(_pack_body, N, GB),
        grid=(1,),
        in_specs=[pl.BlockSpec((2, EB, 128), lambda i: (0, 0, 0)),
                  pl.BlockSpec((EB, 128), lambda i: (0, 0))],
        out_specs=pl.BlockSpec((EB, 128), lambda i: (0, 0)),
        out_shape=jax.ShapeDtypeStruct((EB, 128), jnp.int32),
    )(edge_index.reshape(2, EB, 128), edge_type.reshape(EB, 128))
    packed = jnp.concatenate([packed.reshape(E),
                              jnp.full((e_pad - E,), pad_val)])

    ZR = (-(-(N + 1) // NS) + 7) // 8 * 8
    N_pad = ZR * NS

    mesh = plsc.VectorSubcoreMesh(
        core_axis_name="c", subcore_axis_name="s", num_cores=NC, num_subcores=NS
    )

    R0, R1 = N0 // NBUF, N1 // NBUF
    REFILL_K = (STAGE - NBUF) // NBUF

    @functools.partial(
        pl.kernel,
        out_type=jax.ShapeDtypeStruct((NC, N_pad, O), jnp.float32),
        mesh=mesh,
        scratch_types=[
            pltpu.VMEM((STAGE * CH,), jnp.int32),
            pltpu.VMEM_SHARED((N_pad, O), jnp.float32),
        ]
        + [pltpu.VMEM((CH, O), jnp.float32)] * NBUF
        + [pltpu.VMEM((2, CH), jnp.int32)] * NBUF
        + [pltpu.SemaphoreType.DMA] * (2 * NBUF),
    )
    def sc_scatter(xw_hbm, idx_hbm, out_hbm, *s):
        slab, acc = s[0], s[1]
        rows = s[2:2 + NBUF]
        gd = s[2 + NBUF:2 + 2 * NBUF]
        gsem = s[2 + 2 * NBUF:2 + 3 * NBUF]
        ssem = s[2 + 3 * NBUF:2 + 4 * NBUF]
        cid = lax.axis_index("c")
        sid = lax.axis_index("s")
        mask = jnp.int32((1 << GB) - 1)
        base_e = jnp.where(cid == FAST_CORE, sid * (N0 * CH),
                           NS * (N0 * CH) + sid * (N1 * CH))

        def unpack(ci, b):
            so = lax.rem(ci, STAGE) * CH
            for l in range(CH // 16):
                v = slab[pl.ds(so + 16 * l, 16)]
                gd[b][0, pl.ds(16 * l, 16)] = v & mask
                gd[b][1, pl.ds(16 * l, 16)] = lax.shift_right_logical(v, GB)

        pltpu.sync_copy(idx_hbm.at[pl.ds(base_e, STAGE * CH)], slab)
        zv = jnp.zeros((16,), jnp.float32)

        def zrow(i, carry):
            for l in range(O // 16):
                rows[0][i, pl.ds(16 * l, 16)] = zv
            return carry

        lax.fori_loop(0, CH, zrow, 0)
        nfull, rem_rows = ZR // CH, ZR % CH
        for z in range(nfull):
            pltpu.sync_copy(rows[0], acc.at[pl.ds(sid * ZR + z * CH, CH)])
        if rem_rows:
            pltpu.sync_copy(rows[0].at[pl.ds(0, rem_rows)],
                            acc.at[pl.ds(sid * ZR + nfull * CH, rem_rows)])
        kper = STAGE // NBUF

        def run(table, rounds):
            for b in range(NBUF):
                unpack(b, b)
                pltpu.async_copy(table.at[gd[b].at[0]], rows[b], gsem[b])
            plsc.subcore_barrier()

            def body(k, carry):
                @pl.when(jnp.logical_and(lax.rem(k, kper) == REFILL_K,
                                         k < rounds - 1))
                def _():
                    stg = k // kper + 1
                    pltpu.sync_copy(
                        idx_hbm.at[pl.ds(base_e + stg * (STAGE * CH),
                                         STAGE * CH)], slab)

                for b in range(NBUF):
                    ci = k * NBUF + b
                    pltpu.make_async_copy(table.at[gd[b].at[0]],
                                          rows[b], gsem[b]).wait()
                    pltpu.sync_copy(rows[b], acc.at[gd[b].at[1]], add=True)

                    @pl.when(k < rounds - 1)
                    def _():
                        unpack(ci + NBUF, b)
                        pltpu.async_copy(table.at[gd[b].at[0]], rows[b],
                                         gsem[b])
                return carry

            lax.fori_loop(0, rounds, body, 0)

        @pl.when(cid == FAST_CORE)
        def _():
            run(xw_hbm, R0)

        @pl.when(cid != FAST_CORE)
        def _():
            run(xw_hbm, R1)

        plsc.subcore_barrier()
        pltpu.sync_copy(acc.at[pl.ds(sid * ZR, ZR)],
                        out_hbm.at[cid, pl.ds(sid * ZR, ZR)])

    partial = sc_scatter(xw, packed)

    BN2 = 1000
    out = pl.pallas_call(
        _add_body,
        grid=(N // BN2,),
        in_specs=[pl.BlockSpec((NC, BN2, O), lambda n: (0, n, 0))],
        out_specs=pl.BlockSpec((BN2, O), lambda n: (n, 0)),
        out_shape=jax.ShapeDtypeStruct((N, O), jnp.float32),
    )(partial)
    return out
